# baseline (speedup 1.0000x reference)
import jax
import jax.numpy as jnp
from jax.experimental import pallas as pl
from jax.experimental.pallas import tpu as pltpu


def _round_up(x, m):
    return (x + m - 1) // m * m


def _fused_kernel(f_ref, w01_ref, w02a_ref, w02b_ref, w12_ref,
                  bd_ref, wf_ref, bf_ref, out_ref):
    D = w02a_ref.shape[0]
    f0 = f_ref[:, 0, :].astype(jnp.bfloat16)
    f1 = f_ref[:, 1, :].astype(jnp.bfloat16)
    f2 = f_ref[:, 2, :].astype(jnp.bfloat16)

    r01 = jnp.dot(f0, w01_ref[:D], preferred_element_type=jnp.float32)
    r01 = r01 + jnp.dot(f1, w01_ref[D:], preferred_element_type=jnp.float32)
    r12 = jnp.dot(f1, w12_ref[:D], preferred_element_type=jnp.float32)
    r12 = r12 + jnp.dot(f2, w12_ref[D:], preferred_element_type=jnp.float32)
    r02 = jnp.dot(f0, w02a_ref[...], preferred_element_type=jnp.float32)
    r02 = r02 + jnp.dot(f2, w02b_ref[...],
                        preferred_element_type=jnp.float32)

    bd = bd_ref[...]
    r01 = (r01 + bd[:, :D]).astype(jnp.bfloat16)
    r02 = (r02 + bd[:, D:2 * D]).astype(jnp.bfloat16)
    r12 = (r12 + bd[:, 2 * D:]).astype(jnp.bfloat16)

    wf = wf_ref[...]
    out = jnp.dot(r01, wf[:D], preferred_element_type=jnp.float32)
    out = out + jnp.dot(r02, wf[D:2 * D], preferred_element_type=jnp.float32)
    out = out + jnp.dot(r12, wf[2 * D:], preferred_element_type=jnp.float32)
    out_ref[...] = out + bf_ref[...]


def kernel(features, wd, bd, wf, bf):
    B, three, D = features.shape
    NF = wf.shape[1]

    w01 = wd[:2 * D, :D].astype(jnp.bfloat16)
    w12 = wd[D:, 2 * D:].astype(jnp.bfloat16)
    w02a = wd[:D, D:2 * D].astype(jnp.bfloat16)
    w02b = wd[2 * D:, D:2 * D].astype(jnp.bfloat16)
    wfb = wf.astype(jnp.bfloat16)

    TB = 512
    B_pad = _round_up(B, TB)
    f = features
    if B_pad != B:
        f = jnp.pad(f, ((0, B_pad - B), (0, 0), (0, 0)))

    compiler_params = pltpu.CompilerParams(
        dimension_semantics=("parallel",),
        vmem_limit_bytes=64 * 1024 * 1024,
    )

    out_pad = pl.pallas_call(
        _fused_kernel,
        out_shape=jax.ShapeDtypeStruct((B_pad, NF), jnp.float32),
        grid=(B_pad // TB,),
        in_specs=[
            pl.BlockSpec((TB, 3, D), lambda i: (i, 0, 0)),
            pl.BlockSpec((2 * D, D), lambda i: (0, 0)),
            pl.BlockSpec((D, D), lambda i: (0, 0)),
            pl.BlockSpec((D, D), lambda i: (0, 0)),
            pl.BlockSpec((2 * D, D), lambda i: (0, 0)),
            pl.BlockSpec((1, 3 * D), lambda i: (0, 0)),
            pl.BlockSpec((3 * D, NF), lambda i: (0, 0)),
            pl.BlockSpec((1, NF), lambda i: (0, 0)),
        ],
        out_specs=pl.BlockSpec((TB, NF), lambda i: (i, 0)),
        compiler_params=compiler_params,
    )(f, w01, w02a, w02b, w12, bd, wfb, bf)

    return out_pad[:B, :6].astype(features.dtype)


# speedup vs baseline: 1.2298x; 1.2298x over previous
import jax
import jax.numpy as jnp
from jax.experimental import pallas as pl
from jax.experimental.pallas import tpu as pltpu


def _round_up(x, m):
    return (x + m - 1) // m * m


def _fused_kernel(f_ref, w01_ref, w02a_ref, w02b_ref, w12_ref,
                  bd_ref, wf_ref, bf_ref, out_ref):
    fb = f_ref[...]
    D = w02a_ref.shape[0]

    r01 = jnp.dot(fb[:, :2 * D], w01_ref[...],
                  preferred_element_type=jnp.float32)
    r12 = jnp.dot(fb[:, D:], w12_ref[...],
                  preferred_element_type=jnp.float32)
    r02 = jnp.dot(fb[:, :D], w02a_ref[...],
                  preferred_element_type=jnp.float32)
    r02 = r02 + jnp.dot(fb[:, 2 * D:], w02b_ref[...],
                        preferred_element_type=jnp.float32)

    bd = bd_ref[...]
    r01 = (r01 + bd[:, :D]).astype(jnp.bfloat16)
    r02 = (r02 + bd[:, D:2 * D]).astype(jnp.bfloat16)
    r12 = (r12 + bd[:, 2 * D:]).astype(jnp.bfloat16)

    wf = wf_ref[...]
    out = jnp.dot(r01, wf[:D], preferred_element_type=jnp.float32)
    out = out + jnp.dot(r02, wf[D:2 * D], preferred_element_type=jnp.float32)
    out = out + jnp.dot(r12, wf[2 * D:], preferred_element_type=jnp.float32)
    out_ref[...] = out + bf_ref[...]


def kernel(features, wd, bd, wf, bf):
    B, three, D = features.shape
    NF = wf.shape[1]

    w01 = wd[:2 * D, :D].astype(jnp.bfloat16)
    w12 = wd[D:, 2 * D:].astype(jnp.bfloat16)
    w02a = wd[:D, D:2 * D].astype(jnp.bfloat16)
    w02b = wd[2 * D:, D:2 * D].astype(jnp.bfloat16)
    wfb = wf.astype(jnp.bfloat16)

    TB = 512
    B_pad = _round_up(B, TB)
    f = features.reshape(B, 3 * D).astype(jnp.bfloat16)
    if B_pad != B:
        f = jnp.pad(f, ((0, B_pad - B), (0, 0)))

    compiler_params = pltpu.CompilerParams(
        dimension_semantics=("parallel",),
        vmem_limit_bytes=64 * 1024 * 1024,
    )

    out_pad = pl.pallas_call(
        _fused_kernel,
        out_shape=jax.ShapeDtypeStruct((B_pad, NF), jnp.float32),
        grid=(B_pad // TB,),
        in_specs=[
            pl.BlockSpec((TB, 3 * D), lambda i: (i, 0)),
            pl.BlockSpec((2 * D, D), lambda i: (0, 0)),
            pl.BlockSpec((D, D), lambda i: (0, 0)),
            pl.BlockSpec((D, D), lambda i: (0, 0)),
            pl.BlockSpec((2 * D, D), lambda i: (0, 0)),
            pl.BlockSpec((1, 3 * D), lambda i: (0, 0)),
            pl.BlockSpec((3 * D, NF), lambda i: (0, 0)),
            pl.BlockSpec((1, NF), lambda i: (0, 0)),
        ],
        out_specs=pl.BlockSpec((TB, NF), lambda i: (i, 0)),
        compiler_params=compiler_params,
    )(f, w01, w02a, w02b, w12, bd, wfb, bf)

    return out_pad[:B, :6].astype(features.dtype)


# speedup vs baseline: 1.2387x; 1.0072x over previous
import jax
import jax.numpy as jnp
from jax.experimental import pallas as pl
from jax.experimental.pallas import tpu as pltpu


def _round_up(x, m):
    return (x + m - 1) // m * m


def _fused_kernel(f_ref, w01_ref, w02a_ref, w02b_ref, w12_ref,
                  bd_ref, wf_ref, bf_ref, out_ref):
    fb = f_ref[...]
    D = w02a_ref.shape[0]

    r01 = jnp.dot(fb[:, :2 * D], w01_ref[...],
                  preferred_element_type=jnp.float32)
    r12 = jnp.dot(fb[:, D:], w12_ref[...],
                  preferred_element_type=jnp.float32)
    r02 = jnp.dot(fb[:, :D], w02a_ref[...],
                  preferred_element_type=jnp.float32)
    r02 = r02 + jnp.dot(fb[:, 2 * D:], w02b_ref[...],
                        preferred_element_type=jnp.float32)

    bd = bd_ref[...]
    r01 = r01 + bd[:, :D]
    r02 = r02 + bd[:, D:2 * D]
    r12 = r12 + bd[:, 2 * D:]

    wf = wf_ref[...]
    out = jnp.dot(r01, wf[:D], preferred_element_type=jnp.float32)
    out = out + jnp.dot(r02, wf[D:2 * D], preferred_element_type=jnp.float32)
    out = out + jnp.dot(r12, wf[2 * D:], preferred_element_type=jnp.float32)
    out_ref[...] = out + bf_ref[...]


def kernel(features, wd, bd, wf, bf):
    B, three, D = features.shape
    NF = wf.shape[1]

    w01 = wd[:2 * D, :D].astype(jnp.bfloat16)
    w12 = wd[D:, 2 * D:].astype(jnp.bfloat16)
    w02a = wd[:D, D:2 * D].astype(jnp.bfloat16)
    w02b = wd[2 * D:, D:2 * D].astype(jnp.bfloat16)

    TB = 1024
    B_pad = _round_up(B, TB)
    f = features.reshape(B, 3 * D).astype(jnp.bfloat16)
    if B_pad != B:
        f = jnp.pad(f, ((0, B_pad - B), (0, 0)))

    compiler_params = pltpu.CompilerParams(
        dimension_semantics=("parallel",),
        vmem_limit_bytes=64 * 1024 * 1024,
    )

    out_pad = pl.pallas_call(
        _fused_kernel,
        out_shape=jax.ShapeDtypeStruct((B_pad, NF), jnp.float32),
        grid=(B_pad // TB,),
        in_specs=[
            pl.BlockSpec((TB, 3 * D), lambda i: (i, 0)),
            pl.BlockSpec((2 * D, D), lambda i: (0, 0)),
            pl.BlockSpec((D, D), lambda i: (0, 0)),
            pl.BlockSpec((D, D), lambda i: (0, 0)),
            pl.BlockSpec((2 * D, D), lambda i: (0, 0)),
            pl.BlockSpec((1, 3 * D), lambda i: (0, 0)),
            pl.BlockSpec((3 * D, NF), lambda i: (0, 0)),
            pl.BlockSpec((1, NF), lambda i: (0, 0)),
        ],
        out_specs=pl.BlockSpec((TB, NF), lambda i: (i, 0)),
        compiler_params=compiler_params,
    )(f, w01, w02a, w02b, w12, bd, wf, bf)

    return out_pad[:B, :6].astype(features.dtype)


# speedup vs baseline: 1.5231x; 1.2296x over previous
import jax
import jax.numpy as jnp
from jax.experimental import pallas as pl
from jax.experimental.pallas import tpu as pltpu


def _round_up(x, m):
    return (x + m - 1) // m * m


def _fused_kernel(f0_ref, f1_ref, f2_ref, w01_ref, w02a_ref, w02b_ref,
                  w12_ref, bd_ref, wf_ref, bf_ref, out_ref):
    D = w02a_ref.shape[0]
    f0 = f0_ref[...]
    f1 = f1_ref[...]
    f2 = f2_ref[...]

    r01 = jnp.dot(f0, w01_ref[:D], preferred_element_type=jnp.float32)
    r01 = r01 + jnp.dot(f1, w01_ref[D:], preferred_element_type=jnp.float32)
    r12 = jnp.dot(f1, w12_ref[:D], preferred_element_type=jnp.float32)
    r12 = r12 + jnp.dot(f2, w12_ref[D:], preferred_element_type=jnp.float32)
    r02 = jnp.dot(f0, w02a_ref[...], preferred_element_type=jnp.float32)
    r02 = r02 + jnp.dot(f2, w02b_ref[...],
                        preferred_element_type=jnp.float32)

    bd = bd_ref[...]
    r01 = r01 + bd[:, :D]
    r02 = r02 + bd[:, D:2 * D]
    r12 = r12 + bd[:, 2 * D:]

    wf = wf_ref[...]
    out = jnp.dot(r01, wf[:D], preferred_element_type=jnp.float32)
    out = out + jnp.dot(r02, wf[D:2 * D], preferred_element_type=jnp.float32)
    out = out + jnp.dot(r12, wf[2 * D:], preferred_element_type=jnp.float32)
    out_ref[...] = out + bf_ref[...]


def kernel(features, wd, bd, wf, bf):
    B, three, D = features.shape
    NF = wf.shape[1]

    w01 = wd[:2 * D, :D].astype(jnp.bfloat16)
    w12 = wd[D:, 2 * D:].astype(jnp.bfloat16)
    w02a = wd[:D, D:2 * D].astype(jnp.bfloat16)
    w02b = wd[2 * D:, D:2 * D].astype(jnp.bfloat16)

    f0 = features[:, 0, :].astype(jnp.bfloat16)
    f1 = features[:, 1, :].astype(jnp.bfloat16)
    f2 = features[:, 2, :].astype(jnp.bfloat16)

    TB = 512
    B_pad = _round_up(B, TB)
    if B_pad != B:
        pad = ((0, B_pad - B), (0, 0))
        f0 = jnp.pad(f0, pad)
        f1 = jnp.pad(f1, pad)
        f2 = jnp.pad(f2, pad)

    compiler_params = pltpu.CompilerParams(
        dimension_semantics=("parallel",),
        vmem_limit_bytes=64 * 1024 * 1024,
    )

    out_pad = pl.pallas_call(
        _fused_kernel,
        out_shape=jax.ShapeDtypeStruct((B_pad, NF), jnp.float32),
        grid=(B_pad // TB,),
        in_specs=[
            pl.BlockSpec((TB, D), lambda i: (i, 0)),
            pl.BlockSpec((TB, D), lambda i: (i, 0)),
            pl.BlockSpec((TB, D), lambda i: (i, 0)),
            pl.BlockSpec((2 * D, D), lambda i: (0, 0)),
            pl.BlockSpec((D, D), lambda i: (0, 0)),
            pl.BlockSpec((D, D), lambda i: (0, 0)),
            pl.BlockSpec((2 * D, D), lambda i: (0, 0)),
            pl.BlockSpec((1, 3 * D), lambda i: (0, 0)),
            pl.BlockSpec((3 * D, NF), lambda i: (0, 0)),
            pl.BlockSpec((1, NF), lambda i: (0, 0)),
        ],
        out_specs=pl.BlockSpec((TB, NF), lambda i: (i, 0)),
        compiler_params=compiler_params,
    )(f0, f1, f2, w01, w02a, w02b, w12, bd, wf, bf)

    return out_pad[:B, :6].astype(features.dtype)
